# baseline (speedup 1.0000x reference)
import functools

import jax
import jax.numpy as jnp
from jax import lax
from jax.experimental import pallas as pl
from jax.experimental.pallas import tpu as pltpu
from jax.experimental.pallas import tpu_sc as plsc

_N_FEAT = 26
_ROWS_PER_FEAT = 100000
_EMB = 32
_CH = 128


def kernel(x, table):
    B, F = x.shape
    total = B * F
    info = plsc.get_sparse_core_info()
    NC, NS = info.num_cores, info.num_subcores
    NW = NC * NS
    per_w = total // NW
    NG = per_w // _CH
    x2d = x.reshape(total // _CH, _CH)

    mesh = plsc.VectorSubcoreMesh(core_axis_name="c", subcore_axis_name="s")

    @functools.partial(
        pl.kernel,
        mesh=mesh,
        out_type=jax.ShapeDtypeStruct((total, _EMB), jnp.float32),
        compiler_params=pltpu.CompilerParams(use_tc_tiling_on_sc=False),
        scratch_types=[
            pltpu.VMEM((NG, _CH), jnp.int32),
            pltpu.VMEM((_CH, _EMB), jnp.float32),
            pltpu.SemaphoreType.DMA,
        ],
    )
    def _emb(x_hbm, table_hbm, out_hbm, idxv, rows, sem):
        wid = lax.axis_index("s") * NC + lax.axis_index("c")
        gbase = wid * NG
        ebase = wid * per_w
        pltpu.sync_copy(x_hbm.at[pl.ds(gbase, NG)], idxv)

        def adjust(g, carry):
            base_g = ebase + g * _CH
            for k in range(_CH // 16):
                pos = base_g + k * 16 + lax.iota(jnp.int32, 16)
                off = lax.rem(pos, _N_FEAT) * _ROWS_PER_FEAT
                sl = pl.ds(k * 16, 16)
                idxv[g, sl] = idxv[g, sl] + off
            return carry

        lax.fori_loop(0, NG, adjust, 0)

        def gather(g, carry):
            pltpu.async_copy(table_hbm.at[idxv.at[g]], rows, sem).wait()
            pltpu.sync_copy(rows, out_hbm.at[pl.ds(ebase + g * _CH, _CH)])
            return carry

        lax.fori_loop(0, NG, gather, 0)

    out = _emb(x2d, table)
    return out.reshape(B, F, _EMB)


# speedup vs baseline: 1.0526x; 1.0526x over previous
import functools

import jax
import jax.numpy as jnp
from jax import lax
from jax.experimental import pallas as pl
from jax.experimental.pallas import tpu as pltpu
from jax.experimental.pallas import tpu_sc as plsc

_F = 26
_E = 32
_V = 2600000
_B = 16384
_TOTAL = _B * _F
_W = 2048
_CH = 128


def _untangle(tT):
    grid = (_V + _W - 1) // _W

    def body(in_ref, out_ref):
        t = jnp.transpose(in_ref[...]).reshape(_W // 4, 4, _E)
        out_ref[...] = jnp.concatenate([t[:, a, :] for a in range(4)], axis=1)

    return pl.pallas_call(
        body,
        out_shape=jax.ShapeDtypeStruct((_V // 4, 128), jnp.float32),
        grid=(grid,),
        in_specs=[pl.BlockSpec((_E, _W), lambda j: (0, j))],
        out_specs=pl.BlockSpec((_W // 4, 128), lambda j: (j, 0)),
    )(tT)


def kernel(x, table):
    info = plsc.get_sparse_core_info()
    NC, NS = info.num_cores, info.num_subcores
    NW = NC * NS
    per_w = _TOTAL // NW
    NG = per_w // _CH

    tT = table.T
    L = _untangle(tT)
    Lr = L.reshape(_V, _E)
    xf = x.T.reshape(_TOTAL // _CH, _CH)

    mesh = plsc.VectorSubcoreMesh(core_axis_name="c", subcore_axis_name="s")

    @functools.partial(
        pl.kernel,
        mesh=mesh,
        out_type=jax.ShapeDtypeStruct((_TOTAL, _E), jnp.float32),
        compiler_params=pltpu.CompilerParams(use_tc_tiling_on_sc=False),
        scratch_types=[
            pltpu.VMEM((NG, _CH), jnp.int32),
            pltpu.VMEM((_CH, _E), jnp.float32),
            pltpu.SemaphoreType.DMA,
        ],
    )
    def _emb(x_hbm, tab_hbm, out_hbm, idxv, rows, sem):
        wid = lax.axis_index("s") * NC + lax.axis_index("c")
        gbase = wid * NG
        pltpu.sync_copy(x_hbm.at[pl.ds(gbase, NG)], idxv)

        def adjust(g, carry):
            off = lax.div(gbase + g, _B // _CH) * (_V // _F)
            for k in range(_CH // 16):
                sl = pl.ds(k * 16, 16)
                idxv[g, sl] = idxv[g, sl] + off
            return carry

        lax.fori_loop(0, NG, adjust, 0)

        def gather(g, carry):
            pltpu.async_copy(tab_hbm.at[idxv.at[g]], rows, sem).wait()
            pltpu.sync_copy(rows, out_hbm.at[pl.ds((gbase + g) * _CH, _CH)])
            return carry

        lax.fori_loop(0, NG, gather, 0)

    G = _emb(xf, Lr)
    G3 = G.reshape(_F, _B, _E)
    return jnp.transpose(G3, (1, 0, 2))


# speedup vs baseline: 2.3859x; 2.2666x over previous
import functools

import jax
import jax.numpy as jnp
from jax import lax
from jax.experimental import pallas as pl
from jax.experimental.pallas import tpu as pltpu
from jax.experimental.pallas import tpu_sc as plsc

_F = 26
_E = 32
_V = 2600000
_B = 16384
_TOTAL = _B * _F
_W = 8192
_Q = _W // 4
_NBLK = (_V + _W - 1) // _W
_VP = _NBLK * _W
_CH = 128


def _untangle(tT):
    def body(i0, i1, i2, i3, out_ref):
        stk = jnp.concatenate([i0[...], i1[...], i2[...], i3[...]], axis=0)
        out_ref[...] = jnp.transpose(stk)

    return pl.pallas_call(
        body,
        out_shape=jax.ShapeDtypeStruct((_VP // 4, 128), jnp.float32),
        grid=(_NBLK,),
        in_specs=[
            pl.BlockSpec(
                (_E, _Q),
                (lambda j, u=u: (0, jnp.minimum(4 * j + u, _V // _Q))),
            )
            for u in range(4)
        ],
        out_specs=pl.BlockSpec((_Q, 128), lambda j: (j, 0)),
    )(tT, tT, tT, tT)


def kernel(x, table):
    info = plsc.get_sparse_core_info()
    NC, NS = info.num_cores, info.num_subcores
    NW = NC * NS
    per_w = _TOTAL // NW
    NG = per_w // _CH

    tT = table.T
    L = _untangle(tT)
    Lr = L.reshape(_VP, _E)
    xf = x.T.reshape(_TOTAL // _CH, _CH)

    mesh = plsc.VectorSubcoreMesh(core_axis_name="c", subcore_axis_name="s")

    @functools.partial(
        pl.kernel,
        mesh=mesh,
        out_type=jax.ShapeDtypeStruct((_TOTAL, _E), jnp.float32),
        compiler_params=pltpu.CompilerParams(use_tc_tiling_on_sc=False),
        scratch_types=[
            pltpu.VMEM((NG, _CH), jnp.int32),
            pltpu.VMEM((_CH, _E), jnp.float32),
            pltpu.SemaphoreType.DMA,
        ],
    )
    def _emb(x_hbm, tab_hbm, out_hbm, idxv, rows, sem):
        wid = lax.axis_index("s") * NC + lax.axis_index("c")
        gbase = wid * NG
        pltpu.sync_copy(x_hbm.at[pl.ds(gbase, NG)], idxv)

        def adjust(g, carry):
            off = lax.div(gbase + g, _B // _CH) * (_V // _F)
            for k in range(_CH // 16):
                sl = pl.ds(k * 16, 16)
                r = idxv[g, sl] + off
                hi = lax.bitwise_and(r, jnp.int32(~(_W - 1)))
                inb = lax.bitwise_and(r, jnp.int32(_W - 1))
                u = lax.shift_right_logical(inb, 11)
                q = lax.bitwise_and(inb, jnp.int32(_Q - 1))
                idxv[g, sl] = lax.bitwise_or(
                    lax.bitwise_or(hi, lax.shift_left(q, 2)), u)
            return carry

        lax.fori_loop(0, NG, adjust, 0)

        def gather(g, carry):
            pltpu.async_copy(tab_hbm.at[idxv.at[g]], rows, sem).wait()
            pltpu.sync_copy(rows, out_hbm.at[pl.ds((gbase + g) * _CH, _CH)])
            return carry

        lax.fori_loop(0, NG, gather, 0)

    G = _emb(xf, Lr)
    G3 = G.reshape(_F, _B, _E)
    return jnp.transpose(G3, (1, 0, 2))


# speedup vs baseline: 3.1099x; 1.3034x over previous
import functools

import jax
import jax.numpy as jnp
from jax import lax
from jax.experimental import pallas as pl
from jax.experimental.pallas import tpu as pltpu
from jax.experimental.pallas import tpu_sc as plsc

_F = 26
_E = 32
_V = 2600000
_B = 16384
_TOTAL = _B * _F
_W = 8192
_Q = _W // 4
_NBLK = (_V + _W - 1) // _W
_VP = _NBLK * _W
_CH = 128


def _untangle(tT):
    def body(i0, i1, i2, i3, out_ref):
        stk = jnp.concatenate([i0[...], i1[...], i2[...], i3[...]], axis=0)
        out_ref[...] = jnp.transpose(stk)

    return pl.pallas_call(
        body,
        out_shape=jax.ShapeDtypeStruct((_VP // 4, 128), jnp.float32),
        grid=(_NBLK,),
        in_specs=[
            pl.BlockSpec(
                (_E, _Q),
                (lambda j, u=u: (0, jnp.minimum(4 * j + u, _V // _Q))),
            )
            for u in range(4)
        ],
        out_specs=pl.BlockSpec((_Q, 128), lambda j: (j, 0)),
    )(tT, tT, tT, tT)


def _repack(gp):
    def body(in_ref, out_ref):
        t = jnp.transpose(in_ref[...])
        out_ref[...] = t.reshape(4, _E, _B // 2)

    return pl.pallas_call(
        body,
        out_shape=jax.ShapeDtypeStruct((_F, _E, _B), jnp.float32),
        grid=(7, 2),
        in_specs=[pl.BlockSpec((_B // 2, 128), lambda m, h: (2 * m + h, 0))],
        out_specs=pl.BlockSpec((4, _E, _B // 2), lambda m, h: (m, 0, h)),
    )(gp)


def kernel(x, table):
    info = plsc.get_sparse_core_info()
    NC, NS = info.num_cores, info.num_subcores
    NW = NC * NS
    per_w = _TOTAL // NW
    NG = per_w // _CH

    tT = table.T
    L = _untangle(tT)
    Lr = L.reshape(_VP, _E)
    xf = x.T.reshape(_TOTAL // _CH, _CH)

    mesh = plsc.VectorSubcoreMesh(core_axis_name="c", subcore_axis_name="s")

    @functools.partial(
        pl.kernel,
        mesh=mesh,
        out_type=jax.ShapeDtypeStruct((7 * _B, 128), jnp.float32),
        compiler_params=pltpu.CompilerParams(use_tc_tiling_on_sc=False),
        scratch_types=[
            pltpu.VMEM((NG, _CH), jnp.int32),
            pltpu.VMEM((_CH, _E), jnp.float32),
            pltpu.SemaphoreType.DMA,
        ],
    )
    def _emb(x_hbm, tab_hbm, out_hbm, idxv, rows, sem):
        wid = lax.axis_index("s") * NC + lax.axis_index("c")
        gbase = wid * NG
        pltpu.sync_copy(x_hbm.at[pl.ds(gbase, NG)], idxv)

        def adjust(g, carry):
            off = lax.div(gbase + g, _B // _CH) * (_V // _F)
            for k in range(_CH // 16):
                sl = pl.ds(k * 16, 16)
                r = idxv[g, sl] + off
                hi = lax.bitwise_and(r, jnp.int32(~(_W - 1)))
                inb = lax.bitwise_and(r, jnp.int32(_W - 1))
                u = lax.shift_right_logical(inb, 11)
                q = lax.bitwise_and(inb, jnp.int32(_Q - 1))
                idxv[g, sl] = lax.bitwise_or(
                    lax.bitwise_or(hi, lax.shift_left(q, 2)), u)
            return carry

        lax.fori_loop(0, NG, adjust, 0)

        def gather(g, carry):
            R = gbase + g
            f = lax.div(R, _B // _CH)
            m = lax.div(f, 4)
            v = lax.rem(f, 4)
            b0 = lax.rem(R, _B // _CH) * _CH
            pltpu.async_copy(tab_hbm.at[idxv.at[g]], rows, sem).wait()
            pltpu.sync_copy(
                rows,
                out_hbm.at[pl.ds(m * _B + b0, _CH), pl.ds(32 * v, 32)])
            return carry

        lax.fori_loop(0, NG, gather, 0)

    Gp = _emb(xf, Lr)
    Y = _repack(Gp)
    return jnp.transpose(Y, (2, 0, 1))


# speedup vs baseline: 4.5994x; 1.4790x over previous
import functools

import jax
import jax.numpy as jnp
from jax import lax
from jax.experimental import pallas as pl
from jax.experimental.pallas import tpu as pltpu
from jax.experimental.pallas import tpu_sc as plsc

_F = 26
_E = 32
_V = 2600000
_B = 16384
_TOTAL = _B * _F
_W = 16384
_Q = _W // 4
_NBLK = (_V + _W - 1) // _W
_VP = _NBLK * _W
_CH = 128


def _untangle(tT):
    def body(i0, i1, i2, i3, out_ref):
        stk = jnp.concatenate([i0[...], i1[...], i2[...], i3[...]], axis=0)
        out_ref[...] = jnp.transpose(stk)

    return pl.pallas_call(
        body,
        out_shape=jax.ShapeDtypeStruct((_VP // 4, 128), jnp.float32),
        grid=(_NBLK,),
        in_specs=[
            pl.BlockSpec(
                (_E, _Q),
                (lambda j, u=u: (0, jnp.minimum(4 * j + u, _V // _Q))),
            )
            for u in range(4)
        ],
        out_specs=pl.BlockSpec((_Q, 128), lambda j: (j, 0)),
    )(tT, tT, tT, tT)


def _repack(gp):
    def body(in_ref, out_ref):
        t = jnp.transpose(in_ref[...])
        out_ref[...] = t.reshape(4, _E, _B // 2)

    return pl.pallas_call(
        body,
        out_shape=jax.ShapeDtypeStruct((_F, _E, _B), jnp.float32),
        grid=(7, 2),
        in_specs=[pl.BlockSpec((_B // 2, 128), lambda m, h: (2 * m + h, 0))],
        out_specs=pl.BlockSpec((4, _E, _B // 2), lambda m, h: (m, 0, h)),
    )(gp)


def kernel(x, table):
    info = plsc.get_sparse_core_info()
    NC, NS = info.num_cores, info.num_subcores
    NW = NC * NS
    per_w = _TOTAL // NW
    NG = per_w // _CH

    tT = table.T
    L = _untangle(tT)
    Lr = L.reshape(_VP, _E)
    xf = x.T.reshape(_TOTAL // _CH, _CH)

    mesh = plsc.VectorSubcoreMesh(core_axis_name="c", subcore_axis_name="s")

    @functools.partial(
        pl.kernel,
        mesh=mesh,
        out_type=jax.ShapeDtypeStruct((7 * _B, 128), jnp.float32),
        compiler_params=pltpu.CompilerParams(use_tc_tiling_on_sc=False),
        scratch_types=[
            pltpu.VMEM((NG, _CH), jnp.int32),
            pltpu.VMEM((2, 8 * _CH, _E), jnp.float32),
            pltpu.SemaphoreType.DMA,
            pltpu.SemaphoreType.DMA,
        ],
    )
    def _emb(x_hbm, tab_hbm, out_hbm, idxv, rows2, semg, semw):
        wid = lax.axis_index("s") * NC + lax.axis_index("c")
        gbase = wid * NG
        pltpu.sync_copy(x_hbm.at[pl.ds(gbase, NG)], idxv)

        def adjust(g, carry):
            off = lax.div(gbase + g, _B // _CH) * (_V // _F)
            for k in range(_CH // 16):
                sl = pl.ds(k * 16, 16)
                r = idxv[g, sl] + off
                hi = lax.bitwise_and(r, jnp.int32(~(_W - 1)))
                inb = lax.bitwise_and(r, jnp.int32(_W - 1))
                u = lax.shift_right_logical(inb, 12)
                q = lax.bitwise_and(inb, jnp.int32(_Q - 1))
                idxv[g, sl] = lax.bitwise_or(
                    lax.bitwise_or(hi, lax.shift_left(q, 2)), u)
            return carry

        lax.fori_loop(0, NG, adjust, 0)

        K = 8
        NBATCH = NG // K

        def _wb_window(g):
            R = gbase + g
            f = lax.div(R, _B // _CH)
            m = lax.div(f, 4)
            v = lax.rem(f, 4)
            b0 = lax.rem(R, _B // _CH) * _CH
            return out_hbm.at[pl.ds(m * _B + b0, _CH), pl.ds(32 * v, 32)]

        def batch(n, carry):
            buf = lax.rem(n, 2)

            @pl.when(n >= 2)
            def _drain_prev():
                for _ in range(K):
                    pltpu.make_async_copy(
                        rows2.at[0, pl.ds(0, _CH)], _wb_window(0), semw
                    ).wait()

            cps = [
                pltpu.async_copy(
                    tab_hbm.at[idxv.at[n * K + j]],
                    rows2.at[buf, pl.ds(j * _CH, _CH)],
                    semg,
                )
                for j in range(K)
            ]
            for c in cps:
                c.wait()
            for j in range(K):
                pltpu.async_copy(
                    rows2.at[buf, pl.ds(j * _CH, _CH)],
                    _wb_window(n * K + j),
                    semw,
                )
            return carry

        lax.fori_loop(0, NBATCH, batch, 0)
        for _ in range(2 * K):
            pltpu.make_async_copy(
                rows2.at[0, pl.ds(0, _CH)], _wb_window(0), semw
            ).wait()

    Gp = _emb(xf, Lr)
    Y = _repack(Gp)
    return jnp.transpose(Y, (2, 0, 1))


# speedup vs baseline: 4.6339x; 1.0075x over previous
import functools

import jax
import jax.numpy as jnp
from jax import lax
from jax.experimental import pallas as pl
from jax.experimental.pallas import tpu as pltpu
from jax.experimental.pallas import tpu_sc as plsc

_F = 26
_E = 32
_V = 2600000
_B = 16384
_TOTAL = _B * _F
_W = 16384
_Q = _W // 4
_NBLK = (_V + _W - 1) // _W
_VP = _NBLK * _W
_CH = 128
_NCHUNK = _TOTAL // _CH
_CPF = _B // _CH

_SPLIT_F = 12
_SPLIT_BLK = (_SPLIT_F * (_V // _F) + _W - 1) // _W
_SEG2_BLK = _SPLIT_F * (_V // _F) // _W
_ROW_OFF = _SEG2_BLK * _W


def _untangle(tT, j0, nblk):
    def body(i0, i1, i2, i3, out_ref):
        stk = jnp.concatenate([i0[...], i1[...], i2[...], i3[...]], axis=0)
        out_ref[...] = jnp.transpose(stk)

    return pl.pallas_call(
        body,
        out_shape=jax.ShapeDtypeStruct((nblk * _Q, 128), jnp.float32),
        grid=(nblk,),
        in_specs=[
            pl.BlockSpec(
                (_E, _Q),
                (lambda j, u=u: (0, jnp.minimum(4 * (j0 + j) + u, _V // _Q))),
            )
            for u in range(4)
        ],
        out_specs=pl.BlockSpec((_Q, 128), lambda j: (j, 0)),
    )(tT, tT, tT, tT)


def _make_emb(nchunks_w, base_chunk, base_f, row_off, ngroups):
    mesh = plsc.VectorSubcoreMesh(core_axis_name="c", subcore_axis_name="s")
    K = 8
    NB = nchunks_w // K

    @functools.partial(
        pl.kernel,
        mesh=mesh,
        out_type=jax.ShapeDtypeStruct((ngroups * _B, 128), jnp.float32),
        compiler_params=pltpu.CompilerParams(use_tc_tiling_on_sc=False),
        scratch_types=[
            pltpu.VMEM((nchunks_w, _CH), jnp.int32),
            pltpu.VMEM((2, K * _CH, _E), jnp.float32),
            pltpu.SemaphoreType.DMA,
            pltpu.SemaphoreType.DMA,
        ],
    )
    def _emb(x_hbm, tab_hbm, out_hbm, idxv, rows2, semg, semw):
        info = plsc.get_sparse_core_info()
        NC = info.num_cores
        wid = lax.axis_index("s") * NC + lax.axis_index("c")
        gbase = base_chunk + wid * nchunks_w
        pltpu.sync_copy(x_hbm.at[pl.ds(gbase, nchunks_w)], idxv)

        def adjust(g, carry):
            off = lax.div(gbase + g, _CPF) * (_V // _F)
            for k in range(_CH // 16):
                sl = pl.ds(k * 16, 16)
                r = idxv[g, sl] + off
                hi = lax.bitwise_and(r, jnp.int32(~(_W - 1)))
                inb = lax.bitwise_and(r, jnp.int32(_W - 1))
                u = lax.shift_right_logical(inb, 12)
                q = lax.bitwise_and(inb, jnp.int32(_Q - 1))
                s = lax.bitwise_or(
                    lax.bitwise_or(hi, lax.shift_left(q, 2)), u)
                idxv[g, sl] = s - row_off
            return carry

        lax.fori_loop(0, nchunks_w, adjust, 0)

        def _wb_window(g):
            R = gbase + g
            fl = lax.div(R, _CPF) - base_f
            m = lax.div(fl, 4)
            v = lax.rem(fl, 4)
            b0 = lax.rem(R, _CPF) * _CH
            return out_hbm.at[pl.ds(m * _B + b0, _CH), pl.ds(32 * v, 32)]

        def batch(n, carry):
            buf = lax.rem(n, 2)

            @pl.when(n >= 2)
            def _drain_prev():
                for _ in range(K):
                    pltpu.make_async_copy(
                        rows2.at[0, pl.ds(0, _CH)], _wb_window(0), semw
                    ).wait()

            cps = [
                pltpu.async_copy(
                    tab_hbm.at[idxv.at[n * K + j]],
                    rows2.at[buf, pl.ds(j * _CH, _CH)],
                    semg,
                )
                for j in range(K)
            ]
            for c in cps:
                c.wait()
            for j in range(K):
                pltpu.async_copy(
                    rows2.at[buf, pl.ds(j * _CH, _CH)],
                    _wb_window(n * K + j),
                    semw,
                )
            return carry

        lax.fori_loop(0, NB, batch, 0)
        for _ in range(2 * K):
            pltpu.make_async_copy(
                rows2.at[0, pl.ds(0, _CH)], _wb_window(0), semw
            ).wait()

    return _emb


def _repack(gp1, gp2):
    def body(i1, i2, out_ref):
        m = pl.program_id(0)
        src = jnp.where(m <= 2, i1[...], i2[...])
        out_ref[...] = jnp.transpose(src).reshape(4, _E, _B // 2)

    return pl.pallas_call(
        body,
        out_shape=jax.ShapeDtypeStruct((_F, _E, _B), jnp.float32),
        grid=(7, 2),
        in_specs=[
            pl.BlockSpec(
                (_B // 2, 128),
                lambda m, h: (jnp.minimum(2 * m + h, 5), 0),
            ),
            pl.BlockSpec(
                (_B // 2, 128),
                lambda m, h: (jnp.clip(2 * (m - 3) + h, 0, 7), 0),
            ),
        ],
        out_specs=pl.BlockSpec((4, _E, _B // 2), lambda m, h: (m, 0, h)),
    )(gp1, gp2)


def kernel(x, table):
    NW = 32
    tT = table.T
    xf = x.T.reshape(_NCHUNK, _CH)

    L1 = _untangle(tT, 0, _SPLIT_BLK)
    L2 = _untangle(tT, _SEG2_BLK, _NBLK - _SEG2_BLK)
    L1r = L1.reshape(_SPLIT_BLK * _W, _E)
    L2r = L2.reshape(_VP - _ROW_OFF, _E)

    n1 = _SPLIT_F * _CPF
    emb1 = _make_emb(n1 // NW, 0, 0, 0, _SPLIT_F // 4)
    emb2 = _make_emb((_NCHUNK - n1) // NW, n1, _SPLIT_F, _ROW_OFF,
                     (_F - _SPLIT_F + 3) // 4)

    Gp1 = emb1(xf, L1r)
    Gp2 = emb2(xf, L2r)
    Y = _repack(Gp1, Gp2)
    return jnp.transpose(Y, (2, 0, 1))


# speedup vs baseline: 5.1022x; 1.1011x over previous
import functools

import jax
import jax.numpy as jnp
from jax import lax
from jax.experimental import pallas as pl
from jax.experimental.pallas import tpu as pltpu
from jax.experimental.pallas import tpu_sc as plsc

_F = 26
_E = 32
_V = 2600000
_B = 16384
_TOTAL = _B * _F
_W = 32768
_Q = _W // 4
_NBLK = (_V + _W - 1) // _W
_VP = _NBLK * _W
_CH = 128
_NCHUNK = _TOTAL // _CH
_CPF = _B // _CH

_SPLIT_F = 12
_SPLIT_BLK = (_SPLIT_F * (_V // _F) + _W - 1) // _W
_SEG2_BLK = _SPLIT_F * (_V // _F) // _W
_ROW_OFF = _SEG2_BLK * _W


def _untangle(tT, j0, nblk):
    def body(i0, i1, i2, i3, out_ref):
        stk = jnp.concatenate([i0[...], i1[...], i2[...], i3[...]], axis=0)
        out_ref[...] = jnp.transpose(stk)

    return pl.pallas_call(
        body,
        out_shape=jax.ShapeDtypeStruct((nblk * _Q, 128), jnp.float32),
        grid=(nblk,),
        in_specs=[
            pl.BlockSpec(
                (_E, _Q),
                (lambda j, u=u: (0, jnp.minimum(4 * (j0 + j) + u, _V // _Q))),
            )
            for u in range(4)
        ],
        out_specs=pl.BlockSpec((_Q, 128), lambda j: (j, 0)),
    )(tT, tT, tT, tT)


def _make_emb(nchunks_w, base_chunk, base_f, row_off, ngroups):
    mesh = plsc.VectorSubcoreMesh(core_axis_name="c", subcore_axis_name="s")
    K = 8
    NB = nchunks_w // K

    @functools.partial(
        pl.kernel,
        mesh=mesh,
        out_type=jax.ShapeDtypeStruct((ngroups * _B, 128), jnp.float32),
        compiler_params=pltpu.CompilerParams(use_tc_tiling_on_sc=False),
        scratch_types=[
            pltpu.VMEM((nchunks_w, _CH), jnp.int32),
            pltpu.VMEM((2, K * _CH, _E), jnp.float32),
            pltpu.SemaphoreType.DMA,
            pltpu.SemaphoreType.DMA,
        ],
    )
    def _emb(x_hbm, tab_hbm, out_hbm, idxv, rows2, semg, semw):
        info = plsc.get_sparse_core_info()
        NC = info.num_cores
        wid = lax.axis_index("s") * NC + lax.axis_index("c")
        gbase = base_chunk + wid * nchunks_w
        pltpu.sync_copy(x_hbm.at[pl.ds(gbase, nchunks_w)], idxv)

        def adjust(g, carry):
            off = lax.div(gbase + g, _CPF) * (_V // _F)
            for k in range(_CH // 16):
                sl = pl.ds(k * 16, 16)
                r = idxv[g, sl] + off
                hi = lax.bitwise_and(r, jnp.int32(~(_W - 1)))
                inb = lax.bitwise_and(r, jnp.int32(_W - 1))
                u = lax.shift_right_logical(inb, 13)
                q = lax.bitwise_and(inb, jnp.int32(_Q - 1))
                s = lax.bitwise_or(
                    lax.bitwise_or(hi, lax.shift_left(q, 2)), u)
                idxv[g, sl] = s - row_off
            return carry

        lax.fori_loop(0, nchunks_w, adjust, 0)

        def _wb_window(g):
            R = gbase + g
            fl = lax.div(R, _CPF) - base_f
            m = lax.div(fl, 4)
            v = lax.rem(fl, 4)
            b0 = lax.rem(R, _CPF) * _CH
            return out_hbm.at[pl.ds(m * _B + b0, _CH), pl.ds(32 * v, 32)]

        def batch(n, carry):
            buf = lax.rem(n, 2)

            @pl.when(n >= 2)
            def _drain_prev():
                for _ in range(K):
                    pltpu.make_async_copy(
                        rows2.at[0, pl.ds(0, _CH)], _wb_window(0), semw
                    ).wait()

            cps = [
                pltpu.async_copy(
                    tab_hbm.at[idxv.at[n * K + j]],
                    rows2.at[buf, pl.ds(j * _CH, _CH)],
                    semg,
                )
                for j in range(K)
            ]
            for c in cps:
                c.wait()
            for j in range(K):
                pltpu.async_copy(
                    rows2.at[buf, pl.ds(j * _CH, _CH)],
                    _wb_window(n * K + j),
                    semw,
                )
            return carry

        lax.fori_loop(0, NB, batch, 0)
        for _ in range(2 * K):
            pltpu.make_async_copy(
                rows2.at[0, pl.ds(0, _CH)], _wb_window(0), semw
            ).wait()

    return _emb


def _repack(gp1, gp2):
    def body(i1, i2, out_ref):
        m = pl.program_id(0)
        src = jnp.where(m <= 2, i1[...], i2[...])
        out_ref[...] = jnp.transpose(src).reshape(4, _E, _B // 2)

    return pl.pallas_call(
        body,
        out_shape=jax.ShapeDtypeStruct((_F, _E, _B), jnp.float32),
        grid=(7, 2),
        in_specs=[
            pl.BlockSpec(
                (_B // 2, 128),
                lambda m, h: (jnp.minimum(2 * m + h, 5), 0),
            ),
            pl.BlockSpec(
                (_B // 2, 128),
                lambda m, h: (jnp.clip(2 * (m - 3) + h, 0, 7), 0),
            ),
        ],
        out_specs=pl.BlockSpec((4, _E, _B // 2), lambda m, h: (m, 0, h)),
    )(gp1, gp2)


def kernel(x, table):
    NW = 32
    tT = table.T
    xf = x.T.reshape(_NCHUNK, _CH)

    L1 = _untangle(tT, 0, _SPLIT_BLK)
    L2 = _untangle(tT, _SEG2_BLK, _NBLK - _SEG2_BLK)
    L1r = L1.reshape(_SPLIT_BLK * _W, _E)
    L2r = L2.reshape(_VP - _ROW_OFF, _E)

    n1 = _SPLIT_F * _CPF
    emb1 = _make_emb(n1 // NW, 0, 0, 0, _SPLIT_F // 4)
    emb2 = _make_emb((_NCHUNK - n1) // NW, n1, _SPLIT_F, _ROW_OFF,
                     (_F - _SPLIT_F + 3) // 4)

    Gp1 = emb1(xf, L1r)
    Gp2 = emb2(xf, L2r)
    Y = _repack(Gp1, Gp2)
    return jnp.transpose(Y, (2, 0, 1))


# speedup vs baseline: 5.1372x; 1.0068x over previous
import functools

import jax
import jax.numpy as jnp
from jax import lax
from jax.experimental import pallas as pl
from jax.experimental.pallas import tpu as pltpu
from jax.experimental.pallas import tpu_sc as plsc

_F = 26
_E = 32
_V = 2600000
_B = 16384
_TOTAL = _B * _F
_W = 65536
_Q = _W // 4
_NBLK = (_V + _W - 1) // _W
_VP = _NBLK * _W
_CH = 128
_NCHUNK = _TOTAL // _CH
_CPF = _B // _CH

_SPLIT_F = 12
_SPLIT_BLK = (_SPLIT_F * (_V // _F) + _W - 1) // _W
_SEG2_BLK = _SPLIT_F * (_V // _F) // _W
_ROW_OFF = _SEG2_BLK * _W


def _untangle(tT, j0, nblk):
    def body(i0, i1, i2, i3, out_ref):
        stk = jnp.concatenate([i0[...], i1[...], i2[...], i3[...]], axis=0)
        out_ref[...] = jnp.transpose(stk)

    return pl.pallas_call(
        body,
        out_shape=jax.ShapeDtypeStruct((nblk * _Q, 128), jnp.float32),
        grid=(nblk,),
        in_specs=[
            pl.BlockSpec(
                (_E, _Q),
                (lambda j, u=u: (0, jnp.minimum(4 * (j0 + j) + u, _V // _Q))),
            )
            for u in range(4)
        ],
        out_specs=pl.BlockSpec((_Q, 128), lambda j: (j, 0)),
    )(tT, tT, tT, tT)


def _make_emb(nchunks_w, base_chunk, base_f, row_off, ngroups):
    mesh = plsc.VectorSubcoreMesh(core_axis_name="c", subcore_axis_name="s")
    K = 8
    NB = nchunks_w // K

    @functools.partial(
        pl.kernel,
        mesh=mesh,
        out_type=jax.ShapeDtypeStruct((ngroups * _B, 128), jnp.float32),
        compiler_params=pltpu.CompilerParams(use_tc_tiling_on_sc=False),
        scratch_types=[
            pltpu.VMEM((nchunks_w, _CH), jnp.int32),
            pltpu.VMEM((2, K * _CH, _E), jnp.float32),
            pltpu.SemaphoreType.DMA,
            pltpu.SemaphoreType.DMA,
        ],
    )
    def _emb(x_hbm, tab_hbm, out_hbm, idxv, rows2, semg, semw):
        info = plsc.get_sparse_core_info()
        NC = info.num_cores
        wid = lax.axis_index("s") * NC + lax.axis_index("c")
        gbase = base_chunk + wid * nchunks_w
        pltpu.sync_copy(x_hbm.at[pl.ds(gbase, nchunks_w)], idxv)

        def adjust(g, carry):
            off = lax.div(gbase + g, _CPF) * (_V // _F)
            for k in range(_CH // 16):
                sl = pl.ds(k * 16, 16)
                r = idxv[g, sl] + off
                hi = lax.bitwise_and(r, jnp.int32(~(_W - 1)))
                inb = lax.bitwise_and(r, jnp.int32(_W - 1))
                u = lax.shift_right_logical(inb, 14)
                q = lax.bitwise_and(inb, jnp.int32(_Q - 1))
                s = lax.bitwise_or(
                    lax.bitwise_or(hi, lax.shift_left(q, 2)), u)
                idxv[g, sl] = s - row_off
            return carry

        lax.fori_loop(0, nchunks_w, adjust, 0)

        def _wb_window(g):
            R = gbase + g
            fl = lax.div(R, _CPF) - base_f
            m = lax.div(fl, 4)
            v = lax.rem(fl, 4)
            b0 = lax.rem(R, _CPF) * _CH
            return out_hbm.at[pl.ds(m * _B + b0, _CH), pl.ds(32 * v, 32)]

        def batch(n, carry):
            buf = lax.rem(n, 2)

            @pl.when(n >= 2)
            def _drain_prev():
                for _ in range(K):
                    pltpu.make_async_copy(
                        rows2.at[0, pl.ds(0, _CH)], _wb_window(0), semw
                    ).wait()

            cps = [
                pltpu.async_copy(
                    tab_hbm.at[idxv.at[n * K + j]],
                    rows2.at[buf, pl.ds(j * _CH, _CH)],
                    semg,
                )
                for j in range(K)
            ]
            for c in cps:
                c.wait()
            for j in range(K):
                pltpu.async_copy(
                    rows2.at[buf, pl.ds(j * _CH, _CH)],
                    _wb_window(n * K + j),
                    semw,
                )
            return carry

        lax.fori_loop(0, NB, batch, 0)
        for _ in range(2 * K):
            pltpu.make_async_copy(
                rows2.at[0, pl.ds(0, _CH)], _wb_window(0), semw
            ).wait()

    return _emb


def _repack(gp1, gp2):
    def body(i1, i2, out_ref):
        m = pl.program_id(0)
        src = jnp.where(m <= 2, i1[...], i2[...])
        out_ref[...] = jnp.transpose(src).reshape(4, _E, _B // 2)

    return pl.pallas_call(
        body,
        out_shape=jax.ShapeDtypeStruct((_F, _E, _B), jnp.float32),
        grid=(7, 2),
        in_specs=[
            pl.BlockSpec(
                (_B // 2, 128),
                lambda m, h: (jnp.minimum(2 * m + h, 5), 0),
            ),
            pl.BlockSpec(
                (_B // 2, 128),
                lambda m, h: (jnp.clip(2 * (m - 3) + h, 0, 7), 0),
            ),
        ],
        out_specs=pl.BlockSpec((4, _E, _B // 2), lambda m, h: (m, 0, h)),
    )(gp1, gp2)


def kernel(x, table):
    NW = 32
    tT = table.T
    xf = x.T.reshape(_NCHUNK, _CH)

    L1 = _untangle(tT, 0, _SPLIT_BLK)
    L2 = _untangle(tT, _SEG2_BLK, _NBLK - _SEG2_BLK)
    L1r = L1.reshape(_SPLIT_BLK * _W, _E)
    L2r = L2.reshape(_VP - _ROW_OFF, _E)

    n1 = _SPLIT_F * _CPF
    emb1 = _make_emb(n1 // NW, 0, 0, 0, _SPLIT_F // 4)
    emb2 = _make_emb((_NCHUNK - n1) // NW, n1, _SPLIT_F, _ROW_OFF,
                     (_F - _SPLIT_F + 3) // 4)

    Gp1 = emb1(xf, L1r)
    Gp2 = emb2(xf, L2r)
    Y = _repack(Gp1, Gp2)
    return jnp.transpose(Y, (2, 0, 1))
